# Initial kernel scaffold; baseline (speedup 1.0000x reference)
import jax
import jax.numpy as jnp
from jax.experimental import pallas as pl


def kernel(x):
    raise NotImplementedError("write your pallas kernel here")


# speedup vs baseline: 8.2114x; 8.2114x over previous
import jax
import jax.numpy as jnp
from jax import lax
from jax.experimental import pallas as pl
from jax.experimental.pallas import tpu as pltpu

OUT_LEN = 32


def _conf_pred_kernel(x_ref, conf_ref, pred_ref):
    xb = x_ref[0]
    C = xb.shape[-1]
    m = jnp.max(xb, axis=-1, keepdims=True)
    z = jnp.sum(jnp.exp(xb - m), axis=-1, keepdims=True)
    lane = lax.broadcasted_iota(jnp.int32, xb.shape, 1)
    pidx = jnp.min(jnp.where(xb == m, lane, C), axis=-1, keepdims=True)
    conf_ref[0] = 1.0 / z
    pred_ref[0] = pidx


def _topk_kernel(conf_ref, pred_ref, out_ref, vot_ref):
    conf = conf_ref[...]
    pred = pred_ref[...]
    B, S = conf.shape
    col = lax.broadcasted_iota(jnp.int32, (B, S), 1)

    prev = jnp.concatenate([jnp.full((B, 1), -1, pred.dtype), pred[:, :-1]], axis=1)
    boundary = pred != prev

    a = jnp.where(boundary, col, S)
    nb = jnp.concatenate([a[:, 1:], jnp.full((B, 1), S, a.dtype)], axis=1)
    k = 1
    while k < S:
        shifted = jnp.concatenate(
            [nb[:, k:], jnp.full((B, k), S, nb.dtype)], axis=1)
        nb = jnp.minimum(nb, shifted)
        k *= 2

    voted = jnp.where(boundary, conf * (nb - col).astype(jnp.float32),
                      -jnp.inf)
    vot_ref[...] = voted

    col_out = lax.broadcasted_iota(jnp.int32, (B, OUT_LEN), 1)

    def body(i, acc):
        v = vot_ref[...]
        m = jnp.max(v, axis=1, keepdims=True)
        hit = v == m
        idx = jnp.min(jnp.where(hit, col, S), axis=1, keepdims=True)
        sel = col == idx
        p = jnp.max(jnp.where(sel, pred, 0), axis=1, keepdims=True)
        val = jnp.where(m > -jnp.inf, p.astype(jnp.float32), 0.0)
        vot_ref[...] = jnp.where(sel, -jnp.inf, v)
        return jnp.where(col_out == i, val, acc)

    out_ref[...] = lax.fori_loop(0, OUT_LEN, body, jnp.zeros((B, OUT_LEN),
                                                             jnp.float32))


def kernel(x):
    B, S, C = x.shape
    R = 1024

    conf3, pred3 = pl.pallas_call(
        _conf_pred_kernel,
        grid=(B, S // R),
        in_specs=[pl.BlockSpec((1, R, C), lambda b, s: (b, s, 0))],
        out_specs=[
            pl.BlockSpec((1, R, 1), lambda b, s: (b, s, 0)),
            pl.BlockSpec((1, R, 1), lambda b, s: (b, s, 0)),
        ],
        out_shape=[
            jax.ShapeDtypeStruct((B, S, 1), jnp.float32),
            jax.ShapeDtypeStruct((B, S, 1), jnp.int32),
        ],
    )(x)

    conf = conf3.reshape(B, S)
    pred = pred3.reshape(B, S)

    out = pl.pallas_call(
        _topk_kernel,
        out_shape=jax.ShapeDtypeStruct((B, OUT_LEN), jnp.float32),
        scratch_shapes=[pltpu.VMEM((B, S), jnp.float32)],
    )(conf, pred)

    return out.astype(x.dtype)
